# baseline (speedup 1.0000x reference)
import functools

import jax
import jax.numpy as jnp
from jax import lax
from jax.experimental import pallas as pl
from jax.experimental.pallas import tpu as pltpu
from jax.experimental.pallas import tpu_sc as plsc

_TOTAL = 4 * 4096 * 2048
_NW = 32
_CH = 16384
_PER_W = _TOTAL // _NW
_NCH = _PER_W // _CH
_VECS = _CH // 16
_UNROLL = 8


def _sc_loss_partials(flat_o, flat_t):
    mesh = plsc.VectorSubcoreMesh(core_axis_name="c", subcore_axis_name="s")

    @functools.partial(
        pl.kernel,
        mesh=mesh,
        out_type=jax.ShapeDtypeStruct((_NW, 16), jnp.float32),
        scratch_types=[
            pltpu.VMEM((2, _CH), jnp.float32),
            pltpu.VMEM((2, _CH), jnp.float32),
            pltpu.VMEM((16,), jnp.float32),
            pltpu.SemaphoreType.DMA,
            pltpu.SemaphoreType.DMA,
            pltpu.SemaphoreType.DMA,
            pltpu.SemaphoreType.DMA,
        ],
    )
    def k(o_hbm, t_hbm, out_hbm, o_buf, t_buf, acc_vm, so0, so1, st0, st1):
        wid = lax.axis_index("s") * 2 + lax.axis_index("c")
        base = wid * _PER_W
        sems_o = (so0, so1)
        sems_t = (st0, st1)

        def copy_o(k_idx, b):
            return pltpu.make_async_copy(
                o_hbm.at[pl.ds(base + k_idx * _CH, _CH)], o_buf.at[b], sems_o[b])

        def copy_t(k_idx, b):
            return pltpu.make_async_copy(
                t_hbm.at[pl.ds(base + k_idx * _CH, _CH)], t_buf.at[b], sems_t[b])

        def start(k_idx, b):
            copy_o(k_idx, b).start()
            copy_t(k_idx, b).start()

        def wait(k_idx, b):
            copy_o(k_idx, b).wait()
            copy_t(k_idx, b).wait()

        def chunk_sum(b, acc):
            ob = o_buf.at[b]
            tb = t_buf.at[b]

            def vbody(v, a):
                for u in range(_UNROLL):
                    off = (v * _UNROLL + u) * 16
                    o = ob[pl.ds(off, 16)]
                    t = tb[pl.ds(off, 16)]
                    d = o - t
                    a = a + jnp.where(t != 0.0, d * d, 0.0)
                return a

            return lax.fori_loop(0, _VECS // _UNROLL, vbody, acc)

        start(0, 0)
        start(1, 1)

        def gbody(gg, acc):
            for b in (0, 1):
                k_idx = 2 * gg + b
                wait(k_idx, b)
                acc = chunk_sum(b, acc)
                start(k_idx + 2, b)
            return acc

        acc = lax.fori_loop(0, (_NCH - 2) // 2, gbody,
                            jnp.zeros((16,), jnp.float32))
        for b in (0, 1):
            wait(_NCH - 2 + b, b)
            acc = chunk_sum(b, acc)

        acc_vm[...] = acc
        pltpu.sync_copy(acc_vm, out_hbm.at[wid])

    return k(flat_o, flat_t)


def kernel(output, target):
    flat_o = output.reshape(_TOTAL)
    flat_t = target.reshape(_TOTAL)
    partials = _sc_loss_partials(flat_o, flat_t)
    return jnp.sum(partials) / _TOTAL


# speedup vs baseline: 1.0143x; 1.0143x over previous
import functools

import jax
import jax.numpy as jnp
from jax import lax
from jax.experimental import pallas as pl
from jax.experimental.pallas import tpu as pltpu
from jax.experimental.pallas import tpu_sc as plsc

_TOTAL = 4 * 4096 * 2048
_NW = 32
_CH = 16384
_PER_W = _TOTAL // _NW
_NCH = _PER_W // _CH
_VECS = _CH // 16
_UNROLL = 8


def _sc_loss_partials(flat_o, flat_t):
    mesh = plsc.VectorSubcoreMesh(core_axis_name="c", subcore_axis_name="s")

    @functools.partial(
        pl.kernel,
        mesh=mesh,
        out_type=jax.ShapeDtypeStruct((_NW, 16), jnp.float32),
        scratch_types=[
            pltpu.VMEM((2, _CH), jnp.float32),
            pltpu.VMEM((2, _CH), jnp.float32),
            pltpu.VMEM((16,), jnp.float32),
            pltpu.SemaphoreType.DMA,
            pltpu.SemaphoreType.DMA,
            pltpu.SemaphoreType.DMA,
            pltpu.SemaphoreType.DMA,
        ],
    )
    def k(o_hbm, t_hbm, out_hbm, o_buf, t_buf, acc_vm, so0, so1, st0, st1):
        wid = lax.axis_index("s") * 2 + lax.axis_index("c")
        base = wid * _PER_W
        sems_o = (so0, so1)
        sems_t = (st0, st1)

        def copy_o(k_idx, b):
            return pltpu.make_async_copy(
                o_hbm.at[pl.ds(base + k_idx * _CH, _CH)], o_buf.at[b], sems_o[b])

        def copy_t(k_idx, b):
            return pltpu.make_async_copy(
                t_hbm.at[pl.ds(base + k_idx * _CH, _CH)], t_buf.at[b], sems_t[b])

        def start(k_idx, b):
            copy_o(k_idx, b).start()
            copy_t(k_idx, b).start()

        def wait(k_idx, b):
            copy_o(k_idx, b).wait()
            copy_t(k_idx, b).wait()

        def chunk_sum(b, accs):
            ob = o_buf.at[b]
            tb = t_buf.at[b]

            def vbody(v, a):
                out = []
                for u in range(_UNROLL):
                    off = (v * _UNROLL + u) * 16
                    o = ob[pl.ds(off, 16)]
                    t = tb[pl.ds(off, 16)]
                    d = jnp.where(t != 0.0, o - t, 0.0)
                    out.append(a[u] + d * d)
                return tuple(out)

            return lax.fori_loop(0, _VECS // _UNROLL, vbody, accs)

        start(0, 0)
        start(1, 1)

        def gbody(gg, accs):
            for b in (0, 1):
                k_idx = 2 * gg + b
                wait(k_idx, b)
                accs = chunk_sum(b, accs)
                start(k_idx + 2, b)
            return accs

        zero = jnp.zeros((16,), jnp.float32)
        accs = lax.fori_loop(0, (_NCH - 2) // 2, gbody, (zero,) * _UNROLL)
        for b in (0, 1):
            wait(_NCH - 2 + b, b)
            accs = chunk_sum(b, accs)

        acc = accs[0]
        for u in range(1, _UNROLL):
            acc = acc + accs[u]
        acc_vm[...] = acc
        pltpu.sync_copy(acc_vm, out_hbm.at[wid])

    return k(flat_o, flat_t)


def kernel(output, target):
    flat_o = output.reshape(_TOTAL)
    flat_t = target.reshape(_TOTAL)
    partials = _sc_loss_partials(flat_o, flat_t)
    return jnp.sum(partials) / _TOTAL
